# baseline (speedup 1.0000x reference)
import functools

import jax
import jax.numpy as jnp
from jax import lax
from jax.experimental import pallas as pl
from jax.experimental.pallas import tpu as pltpu
from jax.experimental.pallas import tpu_sc as plsc

B = 16384
D = 64
NEG = 5
NC = 2
NS = 16
NW = NC * NS
ROWS_PER_W = B // NW
C = 128
NCHUNK = ROWS_PER_W // C
L = 16
G = C // L


def _transe_body(h_hbm, r_hbm, t_hbm, tneg_hbm, ent_hbm, rel_hbm,
                 pos_hbm, neg_hbm,
                 hidx, ridx, tidx, nidx,
                 hbuf, rbuf, tbuf, nbuf,
                 pos_s, neg_s, sem):
    wid = lax.axis_index("s") * NC + lax.axis_index("c")
    wbase = wid * ROWS_PER_W

    def chunk_body(cc, carry):
        base = pl.multiple_of(wbase + cc * C, C)

        pltpu.sync_copy(h_hbm.at[pl.ds(base, C)], hidx)
        pltpu.sync_copy(r_hbm.at[pl.ds(base, C)], ridx)
        pltpu.sync_copy(t_hbm.at[pl.ds(base, C)], tidx)
        for j in range(NEG):
            pltpu.sync_copy(tneg_hbm.at[pl.ds(j * B + base, C)],
                            nidx.at[pl.ds(j * C, C)])

        cps = [
            pltpu.async_copy(ent_hbm.at[hidx], hbuf, sem),
            pltpu.async_copy(rel_hbm.at[ridx], rbuf, sem),
            pltpu.async_copy(ent_hbm.at[tidx], tbuf, sem),
        ]
        for j in range(NEG):
            cps.append(pltpu.async_copy(ent_hbm.at[nidx.at[pl.ds(j * C, C)]],
                                        nbuf.at[pl.ds(j * C, C)], sem))
        for cp in cps:
            cp.wait()

        def group_body(g, carry2):
            rows = g * L + lax.iota(jnp.int32, L)
            rows_n = [rows + j * C for j in range(NEG)]
            acc_p = jnp.zeros((L,), jnp.float32)
            acc_n = [jnp.zeros((L,), jnp.float32) for _ in range(NEG)]
            for d in range(D):
                col = jnp.full((L,), d, jnp.int32)
                hv = plsc.load_gather(hbuf, [rows, col])
                rv = plsc.load_gather(rbuf, [rows, col])
                tv = plsc.load_gather(tbuf, [rows, col])
                hr = hv + rv
                acc_p = acc_p + jnp.abs(hr - tv)
                for j in range(NEG):
                    nv = plsc.load_gather(nbuf, [rows_n[j], col])
                    acc_n[j] = acc_n[j] + jnp.abs(hr - nv)
            pos_s[pl.ds(g * L, L)] = acc_p
            for j in range(NEG):
                neg_s[pl.ds(j * C + g * L, L)] = acc_n[j]
            return carry2

        lax.fori_loop(0, G, group_body, 0)

        pltpu.sync_copy(pos_s, pos_hbm.at[pl.ds(base, C)])
        for j in range(NEG):
            pltpu.sync_copy(neg_s.at[pl.ds(j * C, C)],
                            neg_hbm.at[pl.ds(j * B + base, C)])
        return carry

    lax.fori_loop(0, NCHUNK, chunk_body, 0)


_transe_sc = functools.partial(
    pl.kernel,
    out_type=[
        jax.ShapeDtypeStruct((B,), jnp.float32),
        jax.ShapeDtypeStruct((NEG * B,), jnp.float32),
    ],
    mesh=plsc.VectorSubcoreMesh(core_axis_name="c", subcore_axis_name="s"),
    compiler_params=pltpu.CompilerParams(needs_layout_passes=False,
                                         use_tc_tiling_on_sc=False),
    scratch_types=[
        pltpu.VMEM((C,), jnp.int32),
        pltpu.VMEM((C,), jnp.int32),
        pltpu.VMEM((C,), jnp.int32),
        pltpu.VMEM((NEG * C,), jnp.int32),
        pltpu.VMEM((C, D), jnp.float32),
        pltpu.VMEM((C, D), jnp.float32),
        pltpu.VMEM((C, D), jnp.float32),
        pltpu.VMEM((NEG * C, D), jnp.float32),
        pltpu.VMEM((C,), jnp.float32),
        pltpu.VMEM((NEG * C,), jnp.float32),
        pltpu.SemaphoreType.DMA,
    ],
)(_transe_body)


@jax.jit
def kernel(h, r, t, t_neg, entity_emb, relation_emb):
    h = h.astype(jnp.int32)
    r = r.astype(jnp.int32)
    t = t.astype(jnp.int32)
    tneg_t = jnp.transpose(t_neg.astype(jnp.int32)).reshape(NEG * B)
    pos, neg_flat = _transe_sc(h, r, t, tneg_t, entity_emb, relation_emb)
    neg = jnp.transpose(neg_flat.reshape(NEG, B))
    return pos, neg
